# baseline (speedup 1.0000x reference)
import functools

import jax
import jax.numpy as jnp
from jax import lax
from jax.experimental import pallas as pl
from jax.experimental.pallas import tpu as pltpu
from jax.experimental.pallas import tpu_sc as plsc

_CAP = 1_000_000
_DIM = 64
_NW = 32
_RPW = 31248
_CH = 208
_NCH = 150
_TAILMAX = 112
_SCMAX = 31312
_NEG = -3e38


def _sc_scores(keys, qvec):
    mesh = plsc.VectorSubcoreMesh(core_axis_name="c", subcore_axis_name="s")

    @functools.partial(
        pl.kernel,
        mesh=mesh,
        out_type=[
            jax.ShapeDtypeStruct((_NW, 8, 16), jnp.float32),
            jax.ShapeDtypeStruct((_NW, 8, 16), jnp.float32),
            jax.ShapeDtypeStruct((_NW, 8, 16), jnp.int32),
        ],
        scratch_types=[
            pltpu.VMEM((_CH, _DIM), jnp.float32),
            pltpu.VMEM((_CH, _DIM), jnp.float32),
            pltpu.VMEM((_DIM,), jnp.float32),
            pltpu.VMEM((_SCMAX,), jnp.float32),
            pltpu.VMEM((8, 16), jnp.float32),
            pltpu.VMEM((8, 16), jnp.int32),
            pltpu.SemaphoreType.DMA,
            pltpu.SemaphoreType.DMA,
            pltpu.SemaphoreType.DMA,
        ],
        compiler_params=pltpu.CompilerParams(needs_layout_passes=False),
    )
    def sck(keys_hbm, q_hbm, pm_hbm, ps_hbm, pidx_hbm,
            kb0, kb1, qv, scores, st_f, st_i, sem0, sem1, semq):
        cid = lax.axis_index("c")
        sid = lax.axis_index("s")
        wid = sid * 2 + cid
        base = wid * _RPW
        is_last = wid == _NW - 1
        ntail_blk = jnp.where(is_last, 7, 3)
        nblk = jnp.where(is_last, _SCMAX // 16, _RPW // 16)

        pltpu.make_async_copy(q_hbm, qv, semq).start()
        pltpu.make_async_copy(q_hbm, qv, semq).wait()

        def chunk_dma(c, kb, sem):
            return pltpu.make_async_copy(
                keys_hbm.at[pl.ds(base + c * _CH, _CH), :], kb, sem)

        chunk_dma(0, kb0, sem0).start()
        chunk_dma(1, kb1, sem1).start()

        lane = lax.iota(jnp.int32, 16)
        qs = []
        for j in range(_DIM // 16):
            qvj = qv[pl.ds(j * 16, 16)]
            for k in range(16):
                qs.append(qvj[k])

        def compute_chunk(kb, c):
            def blk_body(b, _):
                rowv = b * 16 + lane
                colv = jnp.zeros((16,), jnp.int32)
                acc = jnp.zeros((16,), jnp.float32)
                for d in range(_DIM):
                    kcol = plsc.load_gather(kb, [rowv, colv])
                    acc = acc + kcol * qs[d]
                    if d != _DIM - 1:
                        colv = colv + 1
                scores[pl.ds(c * _CH + b * 16, 16)] = acc
                return 0
            lax.fori_loop(0, _CH // 16, blk_body, 0)

        def pair_body(t, _):
            c0 = t * 2
            chunk_dma(c0, kb0, sem0).wait()
            compute_chunk(kb0, c0)

            @pl.when(c0 + 2 < _NCH)
            def _():
                chunk_dma(c0 + 2, kb0, sem0).start()

            chunk_dma(c0 + 1, kb1, sem1).wait()
            compute_chunk(kb1, c0 + 1)

            @pl.when(c0 + 3 < _NCH)
            def _():
                chunk_dma(c0 + 3, kb1, sem1).start()
            return 0

        lax.fori_loop(0, _NCH // 2, pair_body, 0)

        pltpu.make_async_copy(
            keys_hbm.at[pl.ds(base + _NCH * _CH, _TAILMAX), :],
            kb0.at[pl.ds(0, _TAILMAX), :], sem0).start()
        pltpu.make_async_copy(
            keys_hbm.at[pl.ds(base + _NCH * _CH, _TAILMAX), :],
            kb0.at[pl.ds(0, _TAILMAX), :], sem0).wait()

        def tail_blk(b, _):
            rowv = b * 16 + lane
            colv = jnp.zeros((16,), jnp.int32)
            acc = jnp.zeros((16,), jnp.float32)
            for d in range(_DIM):
                kcol = plsc.load_gather(kb0, [rowv, colv])
                acc = acc + kcol * qs[d]
                if d != _DIM - 1:
                    colv = colv + 1
            scores[pl.ds(_NCH * _CH + b * 16, 16)] = acc
            return 0
        lax.fori_loop(0, ntail_blk, tail_blk, 0)

        def max_body(k, m16):
            return jnp.maximum(m16, scores[pl.ds(k * 16, 16)])
        m16 = lax.fori_loop(0, nblk, max_body, jnp.full((16,), _NEG))
        m = lax.reduce_max_p.bind(m16, axes=(0,))

        big = jnp.int32(2_000_000_000)

        def se_body(k, carry):
            s16, idx16 = carry
            v = scores[pl.ds(k * 16, 16)]
            s16 = s16 + jnp.exp(v - m)
            cand = jnp.where(v >= m, base + k * 16 + lane, big)
            idx16 = jnp.minimum(idx16, cand)
            return (s16, idx16)
        s16, idx16 = lax.fori_loop(
            0, nblk, se_body,
            (jnp.zeros((16,), jnp.float32), jnp.full((16,), big, jnp.int32)))
        s = lax.reduce_sum_p.bind(s16, axes=(0,))
        idx = lax.reduce_min_p.bind(idx16, axes=(0,))

        for j in range(8):
            st_f[j, :] = jnp.full((16,), m, jnp.float32)
        pltpu.sync_copy(st_f, pm_hbm.at[wid])
        for j in range(8):
            st_f[j, :] = jnp.full((16,), s, jnp.float32)
        pltpu.sync_copy(st_f, ps_hbm.at[wid])
        for j in range(8):
            st_i[j, :] = jnp.full((16,), idx, jnp.int32)
        pltpu.sync_copy(st_i, pidx_hbm.at[wid])

    return sck(keys, qvec)


def _comb_body(pm_ref, ps_ref, pidx_ref, idx_out, scale_out):
    pm = pm_ref[...].reshape(_NW * 8, 16)
    ps = ps_ref[...].reshape(_NW * 8, 16)
    pidx = pidx_ref[...].reshape(_NW * 8, 16)
    m = jnp.max(pm, keepdims=True)
    s = jnp.sum(ps * jnp.exp(pm - m), keepdims=True) / 128.0
    big = jnp.int32(2_000_000_000)
    idx = jnp.min(jnp.where(pm >= m, pidx, big), keepdims=True)
    idx_out[...] = jnp.broadcast_to(idx, (1, 16))
    scale_out[...] = 1.0 / s


def _combine(pm, ps, pidx):
    return pl.pallas_call(
        _comb_body,
        out_shape=[
            jax.ShapeDtypeStruct((1, 16), jnp.int32),
            jax.ShapeDtypeStruct((1, 1), jnp.float32),
        ],
    )(pm, ps, pidx)


def _g_body(idx_ref, v_ref, scale_ref, out_ref):
    r = idx_ref[0] % 8
    rows = lax.broadcasted_iota(jnp.int32, (8, _DIM), 0)
    sel = jnp.where(rows == r, v_ref[...], 0.0)
    out_ref[...] = jnp.sum(sel, axis=0, keepdims=True) * scale_ref[0, 0]


def _gather(idx, values, scale):
    grid_spec = pltpu.PrefetchScalarGridSpec(
        num_scalar_prefetch=1,
        grid=(1,),
        in_specs=[
            pl.BlockSpec((8, _DIM), lambda i, idx_ref: (idx_ref[0] // 8, 0)),
            pl.BlockSpec(memory_space=pltpu.SMEM),
        ],
        out_specs=pl.BlockSpec((1, _DIM), lambda i, idx_ref: (0, 0)),
    )
    return pl.pallas_call(
        _g_body,
        grid_spec=grid_spec,
        out_shape=jax.ShapeDtypeStruct((1, _DIM), jnp.float32),
    )(idx, values, scale)


def kernel(query, keys, values):
    pm, ps, pidx = _sc_scores(keys, query)
    idx, scale = _combine(pm, ps, pidx)
    out = _gather(idx.reshape(16), values, scale)
    return out.reshape(_DIM)


# speedup vs baseline: 1.0314x; 1.0314x over previous
import functools

import jax
import jax.numpy as jnp
from jax import lax
from jax.experimental import pallas as pl
from jax.experimental.pallas import tpu as pltpu
from jax.experimental.pallas import tpu_sc as plsc

_CAP = 1_000_000
_DIM = 64
_NW = 32
_RPW = 31248
_CH = 208
_NCH = 150
_TAILMAX = 112
_SCMAX = 31312
_NEG = -3e38


def _sc_scores(keys, qvec):
    mesh = plsc.VectorSubcoreMesh(core_axis_name="c", subcore_axis_name="s")

    @functools.partial(
        pl.kernel,
        mesh=mesh,
        out_type=[
            jax.ShapeDtypeStruct((_NW, 8, 16), jnp.float32),
            jax.ShapeDtypeStruct((_NW, 8, 16), jnp.float32),
            jax.ShapeDtypeStruct((_NW, 8, 16), jnp.int32),
        ],
        scratch_types=[
            pltpu.VMEM((_CH, _DIM), jnp.float32),
            pltpu.VMEM((_CH, _DIM), jnp.float32),
            pltpu.VMEM((_DIM,), jnp.float32),
            pltpu.VMEM((_SCMAX,), jnp.float32),
            pltpu.VMEM((8, 16), jnp.float32),
            pltpu.VMEM((8, 16), jnp.int32),
            pltpu.SemaphoreType.DMA,
            pltpu.SemaphoreType.DMA,
            pltpu.SemaphoreType.DMA,
        ],
        compiler_params=pltpu.CompilerParams(needs_layout_passes=False),
    )
    def sck(keys_hbm, q_hbm, pm_hbm, ps_hbm, pidx_hbm,
            kb0, kb1, qv, scores, st_f, st_i, sem0, sem1, semq):
        cid = lax.axis_index("c")
        sid = lax.axis_index("s")
        wid = sid * 2 + cid
        base = wid * _RPW
        is_last = wid == _NW - 1
        ntail_blk = jnp.where(is_last, 7, 3)
        nblk = jnp.where(is_last, _SCMAX // 16, _RPW // 16)

        pltpu.make_async_copy(q_hbm, qv, semq).start()
        pltpu.make_async_copy(q_hbm, qv, semq).wait()

        def chunk_dma(c, kb, sem):
            return pltpu.make_async_copy(
                keys_hbm.at[pl.ds(base + c * _CH, _CH), :], kb, sem)

        chunk_dma(0, kb0, sem0).start()
        chunk_dma(1, kb1, sem1).start()

        lane = lax.iota(jnp.int32, 16)
        qs = []
        for j in range(_DIM // 16):
            qvj = qv[pl.ds(j * 16, 16)]
            for k in range(16):
                qs.append(qvj[k])

        def compute_chunk(kb, c):
            def blk_body(b, _):
                rowv = b * 16 + lane
                accs = [jnp.zeros((16,), jnp.float32) for _ in range(8)]
                cols = [jnp.full((16,), d0, jnp.int32) for d0 in range(8)]
                for g in range(_DIM // 8):
                    for u in range(8):
                        d = g * 8 + u
                        kcol = plsc.load_gather(kb, [rowv, cols[u]])
                        accs[u] = accs[u] + kcol * qs[d]
                        if g != _DIM // 8 - 1:
                            cols[u] = cols[u] + 8
                acc = ((accs[0] + accs[1]) + (accs[2] + accs[3])) + (
                    (accs[4] + accs[5]) + (accs[6] + accs[7]))
                scores[pl.ds(c * _CH + b * 16, 16)] = acc
                return 0
            lax.fori_loop(0, _CH // 16, blk_body, 0)

        def pair_body(t, _):
            c0 = t * 2
            chunk_dma(c0, kb0, sem0).wait()
            compute_chunk(kb0, c0)

            @pl.when(c0 + 2 < _NCH)
            def _():
                chunk_dma(c0 + 2, kb0, sem0).start()

            chunk_dma(c0 + 1, kb1, sem1).wait()
            compute_chunk(kb1, c0 + 1)

            @pl.when(c0 + 3 < _NCH)
            def _():
                chunk_dma(c0 + 3, kb1, sem1).start()
            return 0

        lax.fori_loop(0, _NCH // 2, pair_body, 0)

        pltpu.make_async_copy(
            keys_hbm.at[pl.ds(base + _NCH * _CH, _TAILMAX), :],
            kb0.at[pl.ds(0, _TAILMAX), :], sem0).start()
        pltpu.make_async_copy(
            keys_hbm.at[pl.ds(base + _NCH * _CH, _TAILMAX), :],
            kb0.at[pl.ds(0, _TAILMAX), :], sem0).wait()

        def tail_blk(b, _):
            rowv = b * 16 + lane
            accs = [jnp.zeros((16,), jnp.float32) for _ in range(8)]
            cols = [jnp.full((16,), d0, jnp.int32) for d0 in range(8)]
            for g in range(_DIM // 8):
                for u in range(8):
                    d = g * 8 + u
                    kcol = plsc.load_gather(kb0, [rowv, cols[u]])
                    accs[u] = accs[u] + kcol * qs[d]
                    if g != _DIM // 8 - 1:
                        cols[u] = cols[u] + 8
            acc = ((accs[0] + accs[1]) + (accs[2] + accs[3])) + (
                (accs[4] + accs[5]) + (accs[6] + accs[7]))
            scores[pl.ds(_NCH * _CH + b * 16, 16)] = acc
            return 0
        lax.fori_loop(0, ntail_blk, tail_blk, 0)

        def max_body(k, m16):
            return jnp.maximum(m16, scores[pl.ds(k * 16, 16)])
        m16 = lax.fori_loop(0, nblk, max_body, jnp.full((16,), _NEG))
        m = lax.reduce_max_p.bind(m16, axes=(0,))

        big = jnp.int32(2_000_000_000)

        def se_body(k, carry):
            s16, idx16 = carry
            v = scores[pl.ds(k * 16, 16)]
            s16 = s16 + jnp.exp(v - m)
            cand = jnp.where(v >= m, base + k * 16 + lane, big)
            idx16 = jnp.minimum(idx16, cand)
            return (s16, idx16)
        s16, idx16 = lax.fori_loop(
            0, nblk, se_body,
            (jnp.zeros((16,), jnp.float32), jnp.full((16,), big, jnp.int32)))
        s = lax.reduce_sum_p.bind(s16, axes=(0,))
        idx = lax.reduce_min_p.bind(idx16, axes=(0,))

        for j in range(8):
            st_f[j, :] = jnp.full((16,), m, jnp.float32)
        pltpu.sync_copy(st_f, pm_hbm.at[wid])
        for j in range(8):
            st_f[j, :] = jnp.full((16,), s, jnp.float32)
        pltpu.sync_copy(st_f, ps_hbm.at[wid])
        for j in range(8):
            st_i[j, :] = jnp.full((16,), idx, jnp.int32)
        pltpu.sync_copy(st_i, pidx_hbm.at[wid])

    return sck(keys, qvec)


def _comb_body(pm_ref, ps_ref, pidx_ref, idx_out, scale_out):
    pm = pm_ref[...].reshape(_NW * 8, 16)
    ps = ps_ref[...].reshape(_NW * 8, 16)
    pidx = pidx_ref[...].reshape(_NW * 8, 16)
    m = jnp.max(pm, keepdims=True)
    s = jnp.sum(ps * jnp.exp(pm - m), keepdims=True) / 128.0
    big = jnp.int32(2_000_000_000)
    idx = jnp.min(jnp.where(pm >= m, pidx, big), keepdims=True)
    idx_out[...] = jnp.broadcast_to(idx, (1, 16))
    scale_out[...] = 1.0 / s


def _combine(pm, ps, pidx):
    return pl.pallas_call(
        _comb_body,
        out_shape=[
            jax.ShapeDtypeStruct((1, 16), jnp.int32),
            jax.ShapeDtypeStruct((1, 1), jnp.float32),
        ],
    )(pm, ps, pidx)


def _g_body(idx_ref, v_ref, scale_ref, out_ref):
    r = idx_ref[0] % 8
    rows = lax.broadcasted_iota(jnp.int32, (8, _DIM), 0)
    sel = jnp.where(rows == r, v_ref[...], 0.0)
    out_ref[...] = jnp.sum(sel, axis=0, keepdims=True) * scale_ref[0, 0]


def _gather(idx, values, scale):
    grid_spec = pltpu.PrefetchScalarGridSpec(
        num_scalar_prefetch=1,
        grid=(1,),
        in_specs=[
            pl.BlockSpec((8, _DIM), lambda i, idx_ref: (idx_ref[0] // 8, 0)),
            pl.BlockSpec(memory_space=pltpu.SMEM),
        ],
        out_specs=pl.BlockSpec((1, _DIM), lambda i, idx_ref: (0, 0)),
    )
    return pl.pallas_call(
        _g_body,
        grid_spec=grid_spec,
        out_shape=jax.ShapeDtypeStruct((1, _DIM), jnp.float32),
    )(idx, values, scale)


def kernel(query, keys, values):
    pm, ps, pidx = _sc_scores(keys, query)
    idx, scale = _combine(pm, ps, pidx)
    out = _gather(idx.reshape(16), values, scale)
    return out.reshape(_DIM)


# speedup vs baseline: 1.3745x; 1.3327x over previous
import jax
import jax.numpy as jnp
from jax import lax
from jax.experimental import pallas as pl
from jax.experimental.pallas import tpu as pltpu

_CAP = 1_000_000
_DIM = 64
_RB = 2000
_NBLK = _CAP // _RB
_NBUF = 10
_OUTER = _NBLK // _NBUF
_NEG = -3e38


def _p1_body(k_hbm, q_ref, idx_out, scale_out, bufs, sems, m_ref, s_ref,
             w_ref, wsc_ref):
    i = pl.program_id(0)
    s = pl.program_id(1)

    def dma(b, c):
        return pltpu.make_async_copy(
            k_hbm.at[pl.ds(b * _RB, _RB), :], bufs.at[c], sems.at[c])

    @pl.when(jnp.logical_and(i == 0, s == 0))
    def _():
        m_ref[...] = jnp.full((1, 1), _NEG, jnp.float32)
        s_ref[...] = jnp.zeros((1, 1), jnp.float32)
        w_ref[...] = jnp.zeros((1, 1), jnp.int32)
        for c in range(_NBUF):
            dma(c, c).start()

    b = i * _NBUF + s
    for c in range(_NBUF):
        @pl.when(s == c)
        def _(c=c):
            dma(b, c).wait()
            a = lax.dot_general(q_ref[...], bufs[c], (((1,), (1,)), ((), ())),
                                preferred_element_type=jnp.float32)
            bm = jnp.max(a, keepdims=True)
            bsum = jnp.sum(jnp.exp(a - bm), keepdims=True)
            m_old = m_ref[...]
            m_new = jnp.maximum(m_old, bm)
            s_ref[...] = (s_ref[...] * jnp.exp(m_old - m_new)
                          + bsum * jnp.exp(bm - m_new))
            m_ref[...] = m_new
            better = bm > m_old
            w_ref[...] = jnp.where(better, jnp.full((1, 1), b, jnp.int32),
                                   w_ref[...])
            wsc_ref[...] = jnp.where(jnp.broadcast_to(better, a.shape), a,
                                     wsc_ref[...])

            @pl.when(b + _NBUF < _NBLK)
            def _():
                dma(b + _NBUF, c).start()

    @pl.when(b == _NBLK - 1)
    def _():
        wsc = wsc_ref[...]
        ids = lax.broadcasted_iota(jnp.int32, wsc.shape, 1)
        big = jnp.int32(2_000_000_000)
        lidx = jnp.min(jnp.where(wsc >= m_ref[...], ids, big), keepdims=True)
        idx_out[...] = jnp.broadcast_to(lidx + w_ref[...] * _RB, (1, 16))
        scale_out[...] = 1.0 / s_ref[...]


def _pass1(keys, qrow):
    return pl.pallas_call(
        _p1_body,
        grid=(_OUTER, _NBUF),
        in_specs=[
            pl.BlockSpec(memory_space=pltpu.HBM),
            pl.BlockSpec((1, _DIM), lambda i, s: (0, 0)),
        ],
        out_specs=[
            pl.BlockSpec((1, 16), lambda i, s: (0, 0)),
            pl.BlockSpec((1, 1), lambda i, s: (0, 0)),
        ],
        out_shape=[
            jax.ShapeDtypeStruct((1, 16), jnp.int32),
            jax.ShapeDtypeStruct((1, 1), jnp.float32),
        ],
        scratch_shapes=[
            pltpu.VMEM((_NBUF, _RB, _DIM), jnp.float32),
            pltpu.SemaphoreType.DMA((_NBUF,)),
            pltpu.VMEM((1, 1), jnp.float32),
            pltpu.VMEM((1, 1), jnp.float32),
            pltpu.VMEM((1, 1), jnp.int32),
            pltpu.VMEM((1, _RB), jnp.float32),
        ],
        compiler_params=pltpu.CompilerParams(
            dimension_semantics=("arbitrary", "arbitrary"),
        ),
    )(keys, qrow)


def _g_body(idx_ref, v_ref, scale_ref, out_ref):
    r = idx_ref[0] % 8
    rows = lax.broadcasted_iota(jnp.int32, (8, _DIM), 0)
    sel = jnp.where(rows == r, v_ref[...], 0.0)
    out_ref[...] = jnp.sum(sel, axis=0, keepdims=True) * scale_ref[0, 0]


def _gather(idx, values, scale):
    grid_spec = pltpu.PrefetchScalarGridSpec(
        num_scalar_prefetch=1,
        grid=(1,),
        in_specs=[
            pl.BlockSpec((8, _DIM), lambda i, idx_ref: (idx_ref[0] // 8, 0)),
            pl.BlockSpec(memory_space=pltpu.SMEM),
        ],
        out_specs=pl.BlockSpec((1, _DIM), lambda i, idx_ref: (0, 0)),
    )
    return pl.pallas_call(
        _g_body,
        grid_spec=grid_spec,
        out_shape=jax.ShapeDtypeStruct((1, _DIM), jnp.float32),
    )(idx, values, scale)


def kernel(query, keys, values):
    qrow = query.reshape(1, _DIM)
    idx, scale = _pass1(keys, qrow)
    out = _gather(idx.reshape(16), values, scale)
    return out.reshape(_DIM)


# speedup vs baseline: 1.5954x; 1.1607x over previous
import functools

import jax
import jax.numpy as jnp
from jax import lax
from jax.experimental import pallas as pl
from jax.experimental.pallas import tpu as pltpu
from jax.experimental.pallas import tpu_sc as plsc

_CAP = 1_000_000
_DIM = 64
_R = 20000
_GRID = _CAP // _R


def _p1_body(k_ref, q_ref, w_out, scale_out, m_ref, s_ref, w_ref):
    i = pl.program_id(0)
    a = lax.dot_general(q_ref[...], k_ref[...], (((1,), (1,)), ((), ())),
                        preferred_element_type=jnp.float32)
    bm = jnp.max(a, keepdims=True)
    bsum = jnp.sum(jnp.exp(a - bm), keepdims=True)

    @pl.when(i == 0)
    def _():
        m_ref[...] = bm
        s_ref[...] = bsum
        w_ref[...] = jnp.zeros((1, 1), jnp.int32)

    @pl.when(i > 0)
    def _():
        m_old = m_ref[...]
        m_new = jnp.maximum(m_old, bm)
        s_ref[...] = s_ref[...] * jnp.exp(m_old - m_new) + bsum * jnp.exp(bm - m_new)
        w_ref[...] = jnp.where(bm > m_old, jnp.full((1, 1), i, jnp.int32), w_ref[...])
        m_ref[...] = m_new

    @pl.when(i == _GRID - 1)
    def _():
        w_out[...] = w_ref[...]
        scale_out[...] = 1.0 / s_ref[...]


def _pass1(keys, qcol):
    return pl.pallas_call(
        _p1_body,
        grid=(_GRID,),
        in_specs=[
            pl.BlockSpec((_R, _DIM), lambda i: (i, 0)),
            pl.BlockSpec((1, _DIM), lambda i: (0, 0)),
        ],
        out_specs=[
            pl.BlockSpec((1, 1), lambda i: (0, 0)),
            pl.BlockSpec((1, 1), lambda i: (0, 0)),
        ],
        out_shape=[
            jax.ShapeDtypeStruct((1, 1), jnp.int32),
            jax.ShapeDtypeStruct((1, 1), jnp.float32),
        ],
        scratch_shapes=[
            pltpu.VMEM((1, 1), jnp.float32),
            pltpu.VMEM((1, 1), jnp.float32),
            pltpu.VMEM((1, 1), jnp.int32),
        ],
        compiler_params=pltpu.CompilerParams(
            dimension_semantics=("arbitrary",),
        ),
    )(keys, qcol)


def _p2_body(w_ref, k_ref, q_ref, idx_out):
    a = jnp.dot(k_ref[...], q_ref[...], preferred_element_type=jnp.float32)
    bm = jnp.max(a, keepdims=True)
    rows = lax.broadcasted_iota(jnp.int32, a.shape, 0)
    big = jnp.int32(2_000_000_000)
    bidx = jnp.min(jnp.where(a >= bm, rows, big), keepdims=True)
    idx_out[...] = jnp.broadcast_to(bidx + w_ref[0] * _R, (1, 16))


def _pass2(w, keys, qcol):
    grid_spec = pltpu.PrefetchScalarGridSpec(
        num_scalar_prefetch=1,
        grid=(1,),
        in_specs=[
            pl.BlockSpec((_R, _DIM), lambda i, w_ref: (w_ref[0], 0)),
            pl.BlockSpec((_DIM, 1), lambda i, w_ref: (0, 0)),
        ],
        out_specs=pl.BlockSpec((1, 16), lambda i, w_ref: (0, 0)),
    )
    return pl.pallas_call(
        _p2_body,
        grid_spec=grid_spec,
        out_shape=jax.ShapeDtypeStruct((1, 16), jnp.int32),
    )(w.reshape(1), keys, qcol)


def _g_body(idx_ref, v_ref, scale_ref, out_ref):
    r = idx_ref[0] % 8
    rows = lax.broadcasted_iota(jnp.int32, (8, _DIM), 0)
    sel = jnp.where(rows == r, v_ref[...], 0.0)
    out_ref[...] = jnp.sum(sel, axis=0, keepdims=True) * scale_ref[0, 0]


def _gather(idx, values, scale):
    grid_spec = pltpu.PrefetchScalarGridSpec(
        num_scalar_prefetch=1,
        grid=(1,),
        in_specs=[
            pl.BlockSpec((8, _DIM), lambda i, idx_ref: (idx_ref[0] // 8, 0)),
            pl.BlockSpec(memory_space=pltpu.SMEM),
        ],
        out_specs=pl.BlockSpec((1, _DIM), lambda i, idx_ref: (0, 0)),
    )
    return pl.pallas_call(
        _g_body,
        grid_spec=grid_spec,
        out_shape=jax.ShapeDtypeStruct((1, _DIM), jnp.float32),
    )(idx, values, scale)


def kernel(query, keys, values):
    qcol = query.reshape(_DIM, 1)
    qrow = query.reshape(1, _DIM)
    w, scale = _pass1(keys, qrow)
    idx = _pass2(w, keys, qcol)
    out = _gather(idx.reshape(16), values, scale)
    return out.reshape(_DIM)
